# Initial kernel scaffold; baseline (speedup 1.0000x reference)
import jax
import jax.numpy as jnp
from jax.experimental import pallas as pl


def kernel(x, table):
    raise NotImplementedError("write your pallas kernel here")


# speedup vs baseline: 3.8679x; 3.8679x over previous
import functools

import jax
import jax.numpy as jnp
import numpy as np
from jax import lax
from jax.experimental import pallas as pl
from jax.experimental.pallas import tpu as pltpu
from jax.experimental.pallas import tpu_sc as plsc

VOCAB = 100000
D_MODEL = 1024
BATCH = 4
SEQ = 2048

_NUM_CORES = 2
_NUM_SUBCORES = 16
_NW = _NUM_CORES * _NUM_SUBCORES

_N = BATCH * SEQ
_B_PER_W = _N // _NW
_CHUNK = 32
_NCHUNK = _B_PER_W // _CHUNK


def _pe_table(seq_len: int, d_model: int) -> np.ndarray:
    pos = np.arange(seq_len, dtype=np.float32)[:, None]
    i = np.arange(0, d_model, 2, dtype=np.float32)[None, :]
    angle = pos / np.power(10000.0, i / d_model)
    pe = np.zeros((seq_len, d_model), dtype=np.float32)
    pe[:, 0::2] = np.sin(angle)
    pe[:, 1::2] = np.cos(angle)
    return pe


_PE = _pe_table(SEQ, D_MODEL)


def _sc_gather(xf, table):
    mesh = plsc.VectorSubcoreMesh(core_axis_name="c", subcore_axis_name="s")

    @functools.partial(
        pl.kernel,
        out_type=jax.ShapeDtypeStruct((_N, D_MODEL), jnp.float32),
        mesh=mesh,
        scratch_types=[
            pltpu.VMEM((_CHUNK,), jnp.int32),
            pltpu.VMEM((_CHUNK,), jnp.int32),
            pltpu.VMEM((_CHUNK, D_MODEL), jnp.float32),
            pltpu.VMEM((_CHUNK, D_MODEL), jnp.float32),
            pltpu.SemaphoreType.DMA,
            pltpu.SemaphoreType.DMA,
            pltpu.SemaphoreType.DMA,
            pltpu.SemaphoreType.DMA,
        ],
    )
    def gather_kernel(xf_hbm, table_hbm, out_hbm, idx0, idx1, rows0, rows1,
                      gsem0, gsem1, osem0, osem1):
        wid = lax.axis_index("s") * _NUM_CORES + lax.axis_index("c")
        base = wid * _B_PER_W

        idx = (idx0, idx1)
        rows = (rows0, rows1)
        gsem = (gsem0, gsem1)
        osem = (osem0, osem1)

        def gather_start(c, b):
            pltpu.sync_copy(xf_hbm.at[pl.ds(base + c * _CHUNK, _CHUNK)],
                            idx[b])
            pltpu.async_copy(table_hbm.at[idx[b]], rows[b], gsem[b])

        def gather_wait(b):
            pltpu.make_async_copy(
                table_hbm.at[idx[b]], rows[b], gsem[b]).wait()

        def out_start(c, b):
            pltpu.async_copy(
                rows[b], out_hbm.at[pl.ds(base + c * _CHUNK, _CHUNK)],
                osem[b])

        def out_wait(b):
            pltpu.make_async_copy(
                rows[b], out_hbm.at[pl.ds(base, _CHUNK)], osem[b]).wait()

        gather_start(0, 0)
        gather_start(1, 1)

        @pl.loop(0, _NCHUNK, step=2)
        def _(c):
            gather_wait(0)
            out_start(c, 0)

            @pl.when(c + 2 < _NCHUNK)
            def _():
                out_wait(0)
                gather_start(c + 2, 0)

            gather_wait(1)
            out_start(c + 1, 1)

            @pl.when(c + 3 < _NCHUNK)
            def _():
                out_wait(1)
                gather_start(c + 3, 1)

        out_wait(0)
        out_wait(1)

    return gather_kernel(xf, table)


_TC_ROWS = 256


def _tc_fixup_body(g_ref, pe_ref, scale_ref, o_ref):
    mask = scale_ref[0, 0, :][:, None]
    o_ref[...] = g_ref[...] * mask + pe_ref[...]


def _tc_fixup(gathered, scale3, pe):
    grid = _N // _TC_ROWS
    return pl.pallas_call(
        _tc_fixup_body,
        grid=(grid,),
        in_specs=[
            pl.BlockSpec((_TC_ROWS, D_MODEL), lambda i: (i, 0)),
            pl.BlockSpec((_TC_ROWS, D_MODEL),
                         lambda i: (i % (SEQ // _TC_ROWS), 0)),
            pl.BlockSpec((1, 1, _TC_ROWS), lambda i: (i, 0, 0)),
        ],
        out_specs=pl.BlockSpec((_TC_ROWS, D_MODEL), lambda i: (i, 0)),
        out_shape=jax.ShapeDtypeStruct((_N, D_MODEL), jnp.float32),
    )(gathered, pe, scale3)


def kernel(x, table):
    xf = x.reshape(_N)
    scale3 = (xf != 1).astype(jnp.float32).reshape(_N // _TC_ROWS, 1, _TC_ROWS)
    pe = jnp.asarray(_PE)
    gathered = _sc_gather(xf, table)
    out = _tc_fixup(gathered, scale3, pe)
    return out.reshape(BATCH, SEQ, D_MODEL)
